# Initial kernel scaffold; baseline (speedup 1.0000x reference)
import jax
import jax.numpy as jnp
from jax.experimental import pallas as pl


def kernel(source_uv_warped, target_uv_pred):
    raise NotImplementedError("write your pallas kernel here")


# speedup vs baseline: 1.5438x; 1.5438x over previous
import functools

import jax
import jax.numpy as jnp
from jax.experimental import pallas as pl

_EPS = 1e-08


def _tc_body(sx_ref, sy_ref, tx_ref, ty_ref, omin_ref, oarg_ref, *, tn, m):
    sx = sx_ref[0]
    sy = sy_ref[0]
    tx = tx_ref[0]
    ty = ty_ref[0]
    dx = jnp.abs(sx - tx) + _EPS
    dy = jnp.abs(sy - ty) + _EPS
    s = dx * dx + dy * dy
    mn = jnp.min(s, axis=1, keepdims=True)
    idx = jax.lax.broadcasted_iota(jnp.int32, (tn, m), 1)
    am = jnp.min(jnp.where(s <= mn, idx, m), axis=1, keepdims=True)
    omin_ref[0] = mn
    oarg_ref[0] = am


def _pairwise_min_tc(sx, sy, tx, ty, *, tn=384, interpret=False):
    b, n, _ = sx.shape
    m = tx.shape[2]
    grid = (b, n // tn)
    src_spec = pl.BlockSpec((1, tn, 1), lambda bi, i: (bi, i, 0))
    tgt_spec = pl.BlockSpec((1, 1, m), lambda bi, i: (bi, 0, 0))
    out_spec = pl.BlockSpec((1, tn, 1), lambda bi, i: (bi, i, 0))
    mn, am = pl.pallas_call(
        functools.partial(_tc_body, tn=tn, m=m),
        grid=grid,
        in_specs=[src_spec, src_spec, tgt_spec, tgt_spec],
        out_specs=[out_spec, out_spec],
        out_shape=[
            jax.ShapeDtypeStruct((b, n, 1), jnp.float32),
            jax.ShapeDtypeStruct((b, n, 1), jnp.int32),
        ],
        interpret=interpret,
    )(sx, sy, tx, ty)
    return mn, am


@jax.jit
def kernel(source_uv_warped, target_uv_pred):
    b = source_uv_warped.shape[0]
    src = jnp.reshape(source_uv_warped, (b, -1, 2))
    tgt = jnp.reshape(target_uv_pred, (b, -1, 2))
    sx = src[:, :, 0:1]
    sy = src[:, :, 1:2]
    tx = tgt[:, :, 0][:, None, :]
    ty = tgt[:, :, 1][:, None, :]
    mn, am = _pairwise_min_tc(sx, sy, tx, ty)
    n = src.shape[1]
    return (jnp.sqrt(mn.reshape(b, n)), am.reshape(b, n))
